# baseline (speedup 1.0000x reference)
import functools

import jax
import jax.numpy as jnp
from jax import lax
from jax.experimental import pallas as pl
from jax.experimental.pallas import tpu as pltpu
from jax.experimental.pallas import tpu_sc as plsc

NC = 2
NS = 16
NW = NC * NS

CHUNK = 128
C0_CHUNKS = 40
C1_CHUNKS = 120
IDXBLK = 8
NBUF = 2
TOT_CHUNKS = NS * (C0_CHUNKS + C1_CHUNKS)
E_PAD = TOT_CHUNKS * CHUNK

N_NODES = 10000
D = 128
AGG_ROWS = 10240
ZERO_ROWS_PER_TILE = AGG_ROWS // NS

TC_BLOCK = 2000


def _pipeline(nchunks, nbuf, start, drain):
  for b in range(nbuf):
    start(b, b)
  steady = (nchunks - nbuf) // nbuf

  @pl.loop(0, steady)
  def _(g):
    for b in range(nbuf):
      j = g * nbuf + b
      drain(j, b)
      start(j + nbuf, b)

  for jj in range(steady * nbuf, nchunks):
    b = jj % nbuf
    drain(jj, b)
    if jj + nbuf < nchunks:
      start(jj + nbuf, b)


def _sc_aggregate(X, ia, ib, zb):
  mesh = plsc.VectorSubcoreMesh(core_axis_name="c", subcore_axis_name="s")

  @functools.partial(
      pl.kernel,
      out_type=jax.ShapeDtypeStruct((NC, AGG_ROWS, D), jnp.float32),
      mesh=mesh,
      scratch_types=[
          pltpu.VMEM((IDXBLK, CHUNK), jnp.int32),
          pltpu.VMEM((IDXBLK, CHUNK), jnp.int32),
          pltpu.VMEM((NBUF, CHUNK, D), jnp.float32),
          pltpu.VMEM_SHARED((AGG_ROWS, D), jnp.float32),
      ] + [pltpu.SemaphoreType.DMA] * NBUF,
  )
  def agg_kernel(x_hbm, ia_hbm, ib_hbm, zb_hbm, out_hbm,
                 ia_v, ib_v, rows_v, agg_sh, *sems):
    cid = lax.axis_index("c")
    sid = lax.axis_index("s")
    wid = cid * NS + sid
    slab = pl.ds(sid * ZERO_ROWS_PER_TILE, ZERO_ROWS_PER_TILE)

    pltpu.sync_copy(zb_hbm, rows_v.at[0])
    for q in range(ZERO_ROWS_PER_TILE // CHUNK):
      pltpu.sync_copy(
          rows_v.at[0],
          agg_sh.at[pl.ds(sid * ZERO_ROWS_PER_TILE + q * CHUNK, CHUNK)])
    plsc.subcore_barrier()

    def start(j, slot):
      pltpu.async_copy(x_hbm.at[ia_v.at[j]], rows_v.at[slot], sems[slot])

    def drain(j, slot):
      pltpu.make_async_copy(
          x_hbm.at[ia_v.at[j]], rows_v.at[slot], sems[slot]).wait()
      pltpu.sync_copy(rows_v.at[slot], agg_sh.at[ib_v.at[j]], add=True)

    def run_side(nchunks, base_rows):
      for blk in range(nchunks // IDXBLK):
        base = base_rows + blk * IDXBLK
        pltpu.sync_copy(ia_hbm.at[pl.ds(base, IDXBLK)], ia_v)
        pltpu.sync_copy(ib_hbm.at[pl.ds(base, IDXBLK)], ib_v)
        _pipeline(IDXBLK, NBUF, start, drain)

    @pl.when(cid == 0)
    def _():
      run_side(C0_CHUNKS, sid * C0_CHUNKS)

    @pl.when(cid == 1)
    def _():
      run_side(C1_CHUNKS, NS * C0_CHUNKS + sid * C1_CHUNKS)

    plsc.subcore_barrier()
    pltpu.sync_copy(agg_sh.at[slab], out_hbm.at[cid, slab])

  return agg_kernel(X, ia, ib, zb)


def _tc_mlp_body(x_ref, agg_ref, wh_ref, bh_ref, wo_ref, bo_ref, o_ref):
  xa = x_ref[...] + agg_ref[0] + agg_ref[1]
  h = lax.dot_general(xa, wh_ref[...], (((1,), (0,)), ((), ())),
                      precision=lax.Precision.HIGHEST,
                      preferred_element_type=jnp.float32)
  h = jnp.maximum(h + bh_ref[...], 0.0)
  o = lax.dot_general(h, wo_ref[...], (((1,), (0,)), ((), ())),
                      precision=lax.Precision.HIGHEST,
                      preferred_element_type=jnp.float32)
  o_ref[...] = jnp.maximum(o + bo_ref[...], 0.0)


def _tc_mlp(X, agg2, W_hidden, b_hidden, W_out, b_out):
  n, d = X.shape
  grid = (n // TC_BLOCK,)
  return pl.pallas_call(
      _tc_mlp_body,
      grid=grid,
      in_specs=[
          pl.BlockSpec((TC_BLOCK, d), lambda i: (i, 0)),
          pl.BlockSpec((NC, TC_BLOCK, d), lambda i: (0, i, 0)),
          pl.BlockSpec(W_hidden.shape, lambda i: (0, 0)),
          pl.BlockSpec((1, d), lambda i: (0, 0)),
          pl.BlockSpec(W_out.shape, lambda i: (0, 0)),
          pl.BlockSpec((1, d), lambda i: (0, 0)),
      ],
      out_specs=pl.BlockSpec((TC_BLOCK, d), lambda i: (i, 0)),
      out_shape=jax.ShapeDtypeStruct((n, W_out.shape[1]), jnp.float32),
  )(X, agg2, W_hidden, b_hidden, W_out, b_out)


def kernel(X, ref_a, ref_b, v_map, v_count, W_hidden, b_hidden, W_out, b_out):
  e = ref_a.shape[0]
  pad = E_PAD - e
  ia = jnp.concatenate(
      [ref_a.astype(jnp.int32), jnp.zeros((pad,), jnp.int32)]
  ).reshape(TOT_CHUNKS, CHUNK)
  dummy = N_NODES + (jnp.arange(pad, dtype=jnp.int32) % (AGG_ROWS - N_NODES))
  ib = jnp.concatenate(
      [ref_b.astype(jnp.int32), dummy]
  ).reshape(TOT_CHUNKS, CHUNK)
  zb = jnp.zeros((CHUNK, D), jnp.float32)

  agg2 = _sc_aggregate(X, ia, ib, zb)
  out = _tc_mlp(X, agg2, W_hidden, b_hidden.reshape(1, -1),
                W_out, b_out.reshape(1, -1))
  return (out, ref_a, ref_b, v_map, v_count)


# speedup vs baseline: 1.1239x; 1.1239x over previous
import functools

import jax
import jax.numpy as jnp
from jax import lax
from jax.experimental import pallas as pl
from jax.experimental.pallas import tpu as pltpu
from jax.experimental.pallas import tpu_sc as plsc

NC = 2
NS = 16
NW = NC * NS

CHUNK = 128
C0_CHUNKS = 120
C1_CHUNKS = 40
IDXBLK = 8
NBUF = 2
TOT_CHUNKS = NS * (C0_CHUNKS + C1_CHUNKS)
E_PAD = TOT_CHUNKS * CHUNK

N_NODES = 10000
D = 128
AGG_ROWS = 10240
ZERO_ROWS_PER_TILE = AGG_ROWS // NS

TC_BLOCK = 2000


def _pipeline(nchunks, nbuf, start, drain):
  for b in range(nbuf):
    start(b, b)
  steady = (nchunks - nbuf) // nbuf

  @pl.loop(0, steady)
  def _(g):
    for b in range(nbuf):
      j = g * nbuf + b
      drain(j, b)
      start(j + nbuf, b)

  for jj in range(steady * nbuf, nchunks):
    b = jj % nbuf
    drain(jj, b)
    if jj + nbuf < nchunks:
      start(jj + nbuf, b)


def _sc_aggregate(X, ia, ib, zb):
  mesh = plsc.VectorSubcoreMesh(core_axis_name="c", subcore_axis_name="s")

  @functools.partial(
      pl.kernel,
      out_type=jax.ShapeDtypeStruct((NC, AGG_ROWS, D), jnp.float32),
      mesh=mesh,
      scratch_types=[
          pltpu.VMEM((IDXBLK, CHUNK), jnp.int32),
          pltpu.VMEM((IDXBLK, CHUNK), jnp.int32),
          pltpu.VMEM((NBUF, CHUNK, D), jnp.float32),
          pltpu.VMEM_SHARED((AGG_ROWS, D), jnp.float32),
      ] + [pltpu.SemaphoreType.DMA] * NBUF,
  )
  def agg_kernel(x_hbm, ia_hbm, ib_hbm, zb_hbm, out_hbm,
                 ia_v, ib_v, rows_v, agg_sh, *sems):
    cid = lax.axis_index("c")
    sid = lax.axis_index("s")
    wid = cid * NS + sid
    slab = pl.ds(sid * ZERO_ROWS_PER_TILE, ZERO_ROWS_PER_TILE)

    pltpu.sync_copy(zb_hbm, rows_v.at[0])
    for q in range(ZERO_ROWS_PER_TILE // CHUNK):
      pltpu.sync_copy(
          rows_v.at[0],
          agg_sh.at[pl.ds(sid * ZERO_ROWS_PER_TILE + q * CHUNK, CHUNK)])
    plsc.subcore_barrier()

    def start(j, slot):
      pltpu.async_copy(x_hbm.at[ia_v.at[j]], rows_v.at[slot], sems[slot])

    def drain(j, slot):
      pltpu.make_async_copy(
          x_hbm.at[ia_v.at[j]], rows_v.at[slot], sems[slot]).wait()
      pltpu.sync_copy(rows_v.at[slot], agg_sh.at[ib_v.at[j]], add=True)

    def run_side(nchunks, base_rows):
      for blk in range(nchunks // IDXBLK):
        base = base_rows + blk * IDXBLK
        pltpu.sync_copy(ia_hbm.at[pl.ds(base, IDXBLK)], ia_v)
        pltpu.sync_copy(ib_hbm.at[pl.ds(base, IDXBLK)], ib_v)
        _pipeline(IDXBLK, NBUF, start, drain)

    @pl.when(cid == 0)
    def _():
      run_side(C0_CHUNKS, sid * C0_CHUNKS)

    @pl.when(cid == 1)
    def _():
      run_side(C1_CHUNKS, NS * C0_CHUNKS + sid * C1_CHUNKS)

    plsc.subcore_barrier()
    pltpu.sync_copy(agg_sh.at[slab], out_hbm.at[cid, slab])

  return agg_kernel(X, ia, ib, zb)


def _tc_mlp_body(x_ref, agg_ref, wh_ref, bh_ref, wo_ref, bo_ref, o_ref):
  xa = x_ref[...] + agg_ref[0] + agg_ref[1]
  h = lax.dot_general(xa, wh_ref[...], (((1,), (0,)), ((), ())),
                      precision=lax.Precision.HIGHEST,
                      preferred_element_type=jnp.float32)
  h = jnp.maximum(h + bh_ref[...], 0.0)
  o = lax.dot_general(h, wo_ref[...], (((1,), (0,)), ((), ())),
                      precision=lax.Precision.HIGHEST,
                      preferred_element_type=jnp.float32)
  o_ref[...] = jnp.maximum(o + bo_ref[...], 0.0)


def _tc_mlp(X, agg2, W_hidden, b_hidden, W_out, b_out):
  n, d = X.shape
  grid = (n // TC_BLOCK,)
  return pl.pallas_call(
      _tc_mlp_body,
      grid=grid,
      in_specs=[
          pl.BlockSpec((TC_BLOCK, d), lambda i: (i, 0)),
          pl.BlockSpec((NC, TC_BLOCK, d), lambda i: (0, i, 0)),
          pl.BlockSpec(W_hidden.shape, lambda i: (0, 0)),
          pl.BlockSpec((1, d), lambda i: (0, 0)),
          pl.BlockSpec(W_out.shape, lambda i: (0, 0)),
          pl.BlockSpec((1, d), lambda i: (0, 0)),
      ],
      out_specs=pl.BlockSpec((TC_BLOCK, d), lambda i: (i, 0)),
      out_shape=jax.ShapeDtypeStruct((n, W_out.shape[1]), jnp.float32),
  )(X, agg2, W_hidden, b_hidden, W_out, b_out)


def kernel(X, ref_a, ref_b, v_map, v_count, W_hidden, b_hidden, W_out, b_out):
  e = ref_a.shape[0]
  pad = E_PAD - e
  ia = jnp.concatenate(
      [ref_a.astype(jnp.int32), jnp.zeros((pad,), jnp.int32)]
  ).reshape(TOT_CHUNKS, CHUNK)
  dummy = N_NODES + (jnp.arange(pad, dtype=jnp.int32) % (AGG_ROWS - N_NODES))
  ib = jnp.concatenate(
      [ref_b.astype(jnp.int32), dummy]
  ).reshape(TOT_CHUNKS, CHUNK)
  zb = jnp.zeros((CHUNK, D), jnp.float32)

  agg2 = _sc_aggregate(X, ia, ib, zb)
  out = _tc_mlp(X, agg2, W_hidden, b_hidden.reshape(1, -1),
                W_out, b_out.reshape(1, -1))
  return (out, ref_a, ref_b, v_map, v_count)
